# Initial kernel scaffold; baseline (speedup 1.0000x reference)
import jax
import jax.numpy as jnp
from jax.experimental import pallas as pl


def kernel(x_nodes, edge_index, eW1, eb1, eW2, eb2, nW1, nb1, nW2, nb2):
    raise NotImplementedError("write your pallas kernel here")


# speedup vs baseline: 3.8885x; 3.8885x over previous
import functools

import jax
import jax.numpy as jnp
from jax import lax
from jax.experimental import pallas as pl
from jax.experimental.pallas import tpu as pltpu
from jax.experimental.pallas import tpu_sc as plsc

N_NODES = 10000
N_EDGES = 320000
H = 128
HH = H // 2

NC = 2
NS = 16
E_PER_W = N_EDGES // NS
C = 80
N_CHUNKS = E_PER_W // C
ROWS_PER_TILE = 624
TAIL_ROWS = N_NODES - NS * ROWS_PER_TILE
DCOLS = 16

_LANES = 16


def _tc_pre_body(x_ref, w_ref, b_ref, pa_ref, qa_ref, pb_ref, qb_ref):
    x = x_ref[...]
    p = jnp.dot(x, w_ref[0:H, :], preferred_element_type=jnp.float32) + b_ref[...]
    q = jnp.dot(x, w_ref[H : 2 * H, :], preferred_element_type=jnp.float32)
    pa_ref[...] = p[:, 0:HH]
    pb_ref[...] = p[:, HH:H]
    qa_ref[...] = q[:, 0:HH]
    qb_ref[...] = q[:, HH:H]


def _tc_pre(x, eW1, eb1):
    blk = 1000
    grid = N_NODES // blk
    half = jax.ShapeDtypeStruct((N_NODES, HH), jnp.float32)
    return pl.pallas_call(
        _tc_pre_body,
        grid=(grid,),
        in_specs=[
            pl.BlockSpec((blk, H), lambda i: (i, 0)),
            pl.BlockSpec((2 * H, H), lambda i: (0, 0)),
            pl.BlockSpec((1, H), lambda i: (0, 0)),
        ],
        out_specs=[pl.BlockSpec((blk, HH), lambda i: (i, 0))] * 4,
        out_shape=[half] * 4,
    )(x, eW1, eb1.reshape(1, H))


def _sc_edge_body(
    pa_hbm, qa_hbm, pb_hbm, qb_hbm, src_hbm, dst_hbm,
    s_out, d_out,
    si_v, di_v, p_v, q_v, ones_v, dz_v, s_sh, d_sh, sem_p, sem_q,
):
    cid = lax.axis_index("c")
    sid = lax.axis_index("s")

    zeros16 = jnp.zeros((_LANES,), jnp.float32)

    def fill_z(i, _):
        for j in range(HH // _LANES):
            p_v[i, pl.ds(j * _LANES, _LANES)] = zeros16
        ones_v[i, pl.ds(0, _LANES)] = jnp.full((_LANES,), 1.0, jnp.float32)
        dz_v[i, pl.ds(0, _LANES)] = zeros16
        return 0

    lax.fori_loop(0, C, fill_z, 0)

    row0 = sid * ROWS_PER_TILE
    for k in range(ROWS_PER_TILE // C):
        pltpu.sync_copy(p_v, s_sh.at[pl.ds(row0 + k * C, C)])
        pltpu.sync_copy(dz_v, d_sh.at[pl.ds(row0 + k * C, C)])
    rem0 = row0 + (ROWS_PER_TILE // C) * C
    rem = ROWS_PER_TILE - (ROWS_PER_TILE // C) * C
    pltpu.sync_copy(p_v.at[pl.ds(0, rem)], s_sh.at[pl.ds(rem0, rem)])
    pltpu.sync_copy(dz_v.at[pl.ds(0, rem)], d_sh.at[pl.ds(rem0, rem)])

    @pl.when(sid == NS - 1)
    def _zero_tail():
        tail0 = NS * ROWS_PER_TILE
        pltpu.sync_copy(p_v.at[pl.ds(0, TAIL_ROWS)], s_sh.at[pl.ds(tail0, TAIL_ROWS)])
        pltpu.sync_copy(dz_v.at[pl.ds(0, TAIL_ROWS)], d_sh.at[pl.ds(tail0, TAIL_ROWS)])

    plsc.subcore_barrier()

    def sweep(p_hbm, q_hbm, do_deg):
        def chunk(k, _):
            base = sid * E_PER_W + k * C
            pltpu.sync_copy(src_hbm.at[pl.ds(base, C)], si_v)
            pltpu.sync_copy(dst_hbm.at[pl.ds(base, C)], di_v)
            cp_p = pltpu.async_copy(p_hbm.at[si_v], p_v, sem_p)
            cp_q = pltpu.async_copy(q_hbm.at[di_v], q_v, sem_q)
            cp_p.wait()
            cp_q.wait()

            def relu_row(i, _):
                for j in range(HH // _LANES):
                    s = pl.ds(j * _LANES, _LANES)
                    q_v[i, s] = jnp.maximum(p_v[i, s] + q_v[i, s], 0.0)
                return 0

            lax.fori_loop(0, C, relu_row, 0)
            pltpu.sync_copy(q_v, s_sh.at[di_v], add=True)
            if do_deg:
                pltpu.sync_copy(ones_v, d_sh.at[di_v], add=True)
            return 0

        lax.fori_loop(0, N_CHUNKS, chunk, 0)

    @pl.when(cid == 0)
    def _sweep_a():
        sweep(pa_hbm, qa_hbm, True)

    @pl.when(cid == 1)
    def _sweep_b():
        sweep(pb_hbm, qb_hbm, False)

    plsc.subcore_barrier()
    pltpu.sync_copy(
        s_sh.at[pl.ds(row0, ROWS_PER_TILE)],
        s_out.at[cid].at[pl.ds(row0, ROWS_PER_TILE)],
    )

    @pl.when(cid == 0)
    def _write_deg():
        pltpu.sync_copy(
            d_sh.at[pl.ds(row0, ROWS_PER_TILE)],
            d_out.at[pl.ds(row0, ROWS_PER_TILE)],
        )

    @pl.when(sid == NS - 1)
    def _write_tail():
        tail0 = NS * ROWS_PER_TILE
        pltpu.sync_copy(
            s_sh.at[pl.ds(tail0, TAIL_ROWS)],
            s_out.at[cid].at[pl.ds(tail0, TAIL_ROWS)],
        )

        @pl.when(cid == 0)
        def _write_deg_tail():
            pltpu.sync_copy(
                d_sh.at[pl.ds(tail0, TAIL_ROWS)],
                d_out.at[pl.ds(tail0, TAIL_ROWS)],
            )


@functools.partial(
    pl.kernel,
    out_type=[
        jax.ShapeDtypeStruct((NC, N_NODES, HH), jnp.float32),
        jax.ShapeDtypeStruct((N_NODES, DCOLS), jnp.float32),
    ],
    mesh=plsc.VectorSubcoreMesh(core_axis_name="c", subcore_axis_name="s"),
    compiler_params=pltpu.CompilerParams(use_tc_tiling_on_sc=False),
    scratch_types=[
        pltpu.VMEM((C,), jnp.int32),
        pltpu.VMEM((C,), jnp.int32),
        pltpu.VMEM((C, HH), jnp.float32),
        pltpu.VMEM((C, HH), jnp.float32),
        pltpu.VMEM((C, DCOLS), jnp.float32),
        pltpu.VMEM((C, DCOLS), jnp.float32),
        pltpu.VMEM_SHARED((N_NODES, HH), jnp.float32),
        pltpu.VMEM_SHARED((N_NODES, DCOLS), jnp.float32),
        pltpu.SemaphoreType.DMA,
        pltpu.SemaphoreType.DMA,
    ],
)
def _sc_edge(*args):
    _sc_edge_body(*args)


def _tc_node_body(
    s_ref, d_ref, x_ref, eW2_ref, eb2_ref, nW1_ref, nb1_ref, nW2_ref, nb2_ref,
    out_ref,
):
    w1b = nW1_ref[H : 2 * H, :]
    m = jnp.dot(eW2_ref[...], w1b, preferred_element_type=jnp.float32)
    v = jnp.dot(eb2_ref[...], w1b, preferred_element_type=jnp.float32)
    s = jnp.concatenate([s_ref[0], s_ref[1]], axis=-1)
    deg = d_ref[...][:, 0:1]
    x = x_ref[...]
    g = (
        jnp.dot(x, nW1_ref[0:H, :], preferred_element_type=jnp.float32)
        + jnp.dot(s, m, preferred_element_type=jnp.float32)
        + deg * v
        + nb1_ref[...]
    )
    g = jnp.maximum(g, 0.0)
    out_ref[...] = (
        jnp.dot(g, nW2_ref[...], preferred_element_type=jnp.float32) + nb2_ref[...]
    )


def _tc_node(S, D, x, eW2, eb2, nW1, nb1, nW2, nb2):
    blk = 1000
    grid = N_NODES // blk
    full = lambda r, c: pl.BlockSpec((r, c), lambda i: (0, 0))
    return pl.pallas_call(
        _tc_node_body,
        grid=(grid,),
        in_specs=[
            pl.BlockSpec((NC, blk, HH), lambda i: (0, i, 0)),
            pl.BlockSpec((blk, DCOLS), lambda i: (i, 0)),
            pl.BlockSpec((blk, H), lambda i: (i, 0)),
            full(H, H),
            full(1, H),
            full(2 * H, H),
            full(1, H),
            full(H, H),
            full(1, H),
        ],
        out_specs=pl.BlockSpec((blk, H), lambda i: (i, 0)),
        out_shape=jax.ShapeDtypeStruct((N_NODES, H), jnp.float32),
    )(
        S, D, x, eW2, eb2.reshape(1, H), nW1, nb1.reshape(1, H), nW2,
        nb2.reshape(1, H),
    )


def kernel(x_nodes, edge_index, eW1, eb1, eW2, eb2, nW1, nb1, nW2, nb2):
    src = edge_index[0]
    dst = edge_index[1]
    pa, qa, pb, qb = _tc_pre(x_nodes, eW1, eb1)
    S, D = _sc_edge(pa, qa, pb, qb, src, dst)
    return _tc_node(S, D, x_nodes, eW2, eb2, nW1, nb1, nW2, nb2)
